# baseline (speedup 1.0000x reference)
import functools

import jax
import jax.numpy as jnp
from jax import lax
from jax.experimental import pallas as pl
from jax.experimental.pallas import tpu as pltpu
from jax.experimental.pallas import tpu_sc as plsc

B, L, D = 4096, 200, 64
POS_V = 2 * 200 + 1
ACT_V = 8 + 1
NC, NS = 2, 16
NW = NC * NS
BPW = B // NW
LCH = 25
PPT = 26
CPT = PPT * ACT_V
CV_PAD = NS * CPT
HPT = CPT // 2
NG = D // 16
TILE = 8 * 128
OST = D // 8 * TILE


def _sc_body(item_idsT, pos_idsT, act_idsT, user_ids,
             item_tab, user_tab, pos_tab, act_tab, idxc_hbm,
             out_hbm,
             combo_sh, pos_loc, act_loc, uid_loc, user_loc, idxc_loc,
             iid_loc, pid_loc, aid_loc, cidx, rows, combo,
             ostage0, ostage1,
             isem0, isem1, csem0, csem1, osem0, osem1, ssem):
    cid = lax.axis_index("c")
    sid = lax.axis_index("s")
    wid = sid * NC + cid
    b0 = wid * BPW

    pltpu.sync_copy(pos_tab.at[pl.ds(sid * PPT, PPT)], pos_loc)
    pltpu.sync_copy(act_tab, act_loc)
    pltpu.sync_copy(user_ids.at[pl.ds(b0, BPW)], uid_loc)
    pltpu.async_copy(user_tab.at[uid_loc], user_loc, ssem).wait()

    pltpu.sync_copy(idxc_hbm, idxc_loc)
    idxc = [idxc_loc[pl.ds(g * 16, 16)] for g in range(NG)]

    avs = [[act_loc[a, pl.ds(g * 16, 16)] for g in range(NG)]
           for a in range(ACT_V)]
    for half in range(2):
        def build_p(pp, carry):
            p = half * (PPT // 2) + pp
            for g in range(NG):
                sl = pl.ds(g * 16, 16)
                pv = pos_loc[p, sl]
                for a in range(ACT_V):
                    rows[0, pp * ACT_V + a, sl] = pv + avs[a][g]
            return carry

        lax.fori_loop(0, PPT // 2, build_p, 0)
        pltpu.sync_copy(rows.at[0, pl.ds(0, HPT)],
                        combo_sh.at[pl.ds(sid * CPT + half * HPT, HPT)])
    plsc.subcore_barrier()

    isems = (isem0, isem1)
    csems = (csem0, csem1)
    osems = (osem0, osem1)
    osts = (ostage0, ostage1)

    def stage_ids(c):
        cpar = lax.rem(c, 2)
        l0 = c * LCH
        pltpu.sync_copy(item_idsT.at[pl.ds(l0, LCH), pl.ds(b0, BPW)],
                        iid_loc.at[cpar])
        pltpu.sync_copy(pos_idsT.at[pl.ds(l0, LCH), pl.ds(b0, BPW)],
                        pid_loc.at[cpar])
        pltpu.sync_copy(act_idsT.at[pl.ds(l0, LCH), pl.ds(b0, BPW)],
                        aid_loc.at[cpar])

    def prep_cidx(l, par):
        cpar = lax.rem(l // LCH, 2)
        lo = lax.rem(l, LCH)
        for g in range(BPW // 16):
            sl = pl.ds(g * 16, 16)
            cidx[par, sl] = (pid_loc[cpar, lo, sl] * ACT_V
                             + aid_loc[cpar, lo, sl])

    def gathers(l, par):
        cpar = lax.rem(l // LCH, 2)
        lo = lax.rem(l, LCH)
        pltpu.async_copy(item_tab.at[iid_loc.at[cpar, lo]],
                         rows.at[par], isems[par])
        pltpu.async_copy(combo_sh.at[cidx.at[par]],
                         combo.at[par], csems[par])

    def wait_gathers(l, par):
        cpar = lax.rem(l // LCH, 2)
        lo = lax.rem(l, LCH)
        pltpu.make_async_copy(item_tab.at[iid_loc.at[cpar, lo]],
                              rows.at[par], isems[par]).wait()
        pltpu.make_async_copy(combo_sh.at[cidx.at[par]],
                              combo.at[par], csems[par]).wait()

    def issue_outs(l, par):
        base = l * (D // 8) * NW + wid
        for dblk in range(D // 8):
            pltpu.async_copy(osts[par].at[pl.ds(dblk * TILE, TILE)],
                             out_hbm.at[base + dblk * NW], osems[par])

    def wait_outs(l, par):
        base = l * (D // 8) * NW + wid
        for dblk in range(D // 8):
            pltpu.make_async_copy(osts[par].at[pl.ds(dblk * TILE, TILE)],
                                  out_hbm.at[base + dblk * NW],
                                  osems[par]).wait()

    def compute(l, par):
        ost = osts[par]

        def row8(i, carry):
            cur = carry
            for j in range(8):
                r = i * 8 + j
                for g in range(NG):
                    sl = pl.ds(g * 16, 16)
                    v = (rows[par, r, sl] + combo[par, r, sl]
                         + user_loc[r, sl])
                    plsc.store_scatter(ost, [cur[g]], v)
                cur = tuple(c + 1 for c in cur)
            return cur

        lax.fori_loop(0, BPW // 8, row8, tuple(idxc))

    stage_ids(0)
    prep_cidx(0, 0)
    gathers(0, 0)

    def loop_body(k, carry):
        for par in range(2):
            l = 2 * k + par
            wait_gathers(l, par)
            nxt = l + 1

            @pl.when(nxt < L)
            def _prep():
                @pl.when(lax.rem(nxt, LCH) == 0)
                def _stage():
                    stage_ids(nxt // LCH)
                prep_cidx(nxt, 1 - par)
                gathers(nxt, 1 - par)

            @pl.when(l >= 2)
            def _wout():
                wait_outs(l - 2, par)

            compute(l, par)
            issue_outs(l, par)
        return carry

    lax.fori_loop(0, L // 2, loop_body, 0)
    wait_outs(L - 2, 0)
    wait_outs(L - 1, 1)


@jax.jit
def _run(item_idsT, pos_idsT, act_idsT, user_ids,
         item_tab, user_tab, pos_tab, act_tab):
    mesh = plsc.VectorSubcoreMesh(core_axis_name="c", subcore_axis_name="s")
    f = functools.partial(
        pl.kernel,
        out_type=jax.ShapeDtypeStruct((L * (D // 8) * NW, TILE),
                                      jnp.float32),
        mesh=mesh,
        compiler_params=pltpu.CompilerParams(use_tc_tiling_on_sc=False,
                                             needs_layout_passes=False),
        scratch_types=[
            pltpu.VMEM_SHARED((CV_PAD, D), jnp.float32),
            pltpu.VMEM((PPT, D), jnp.float32),
            pltpu.VMEM((ACT_V, D), jnp.float32),
            pltpu.VMEM((BPW,), jnp.int32),
            pltpu.VMEM((BPW, D), jnp.float32),
            pltpu.VMEM((D,), jnp.int32),
            pltpu.VMEM((2, LCH, BPW), jnp.int32),
            pltpu.VMEM((2, LCH, BPW), jnp.int32),
            pltpu.VMEM((2, LCH, BPW), jnp.int32),
            pltpu.VMEM((2, BPW), jnp.int32),
            pltpu.VMEM((2, BPW, D), jnp.float32),
            pltpu.VMEM((2, BPW, D), jnp.float32),
            pltpu.VMEM((OST,), jnp.float32),
            pltpu.VMEM((OST,), jnp.float32),
            pltpu.SemaphoreType.DMA,
            pltpu.SemaphoreType.DMA,
            pltpu.SemaphoreType.DMA,
            pltpu.SemaphoreType.DMA,
            pltpu.SemaphoreType.DMA,
            pltpu.SemaphoreType.DMA,
            pltpu.SemaphoreType.DMA,
        ],
    )(_sc_body)
    pos_pad = jnp.zeros((NS * PPT, D), jnp.float32).at[:POS_V].set(pos_tab)
    idxc_arr = jnp.array([(d // 8) * TILE + (d % 8) * 128 for d in range(D)],
                         jnp.int32)
    out = f(item_idsT, pos_idsT, act_idsT, user_ids,
            item_tab, user_tab, pos_pad, act_tab, idxc_arr)
    return (out.reshape(L, D // 8, B // BPW, 8, BPW)
            .transpose(2, 4, 0, 1, 3).reshape(B, L, D))


def kernel(item_ids, pos_ids, action_ids, user_ids,
           item_table, user_table, pos_table, action_table):
    return _run(item_ids.astype(jnp.int32).T,
                pos_ids.astype(jnp.int32).T,
                action_ids.astype(jnp.int32).T,
                user_ids.astype(jnp.int32),
                item_table, user_table, pos_table, action_table)


# speedup vs baseline: 1.7722x; 1.7722x over previous
import functools

import jax
import jax.numpy as jnp
from jax import lax
from jax.experimental import pallas as pl
from jax.experimental.pallas import tpu as pltpu
from jax.experimental.pallas import tpu_sc as plsc

B, L, D = 4096, 200, 64
POS_V = 2 * 200 + 1
ACT_V = 8 + 1
NC, NS = 2, 16
NW = NC * NS
BPW = B // NW
CB = 8
G0, G1 = 104, 96
NG = D // 16
PPT = 26
CPT = PPT * ACT_V
CV_PAD = NS * CPT
HPT = CPT // 2
OW = 128
ORPB = L * D // OW


def _sc_body(item_ids, pos_ids, act_ids, user_rows,
             item_tab, pos_tab, act_tab,
             out_hbm,
             combo_sh, pos_loc, act_loc, user_loc,
             iid_loc, pid_loc, aid_loc, cidx, rows, combo, ostage,
             isem0, isem1, csem0, csem1, osem0, osem1, ssem):
    cid = lax.axis_index("c")
    sid = lax.axis_index("s")
    wid = sid * NC + cid
    base_b = wid * BPW

    pltpu.sync_copy(pos_tab.at[pl.ds(sid * PPT, PPT)], pos_loc)
    pltpu.sync_copy(act_tab, act_loc)
    pltpu.sync_copy(user_rows.at[pl.ds(base_b, BPW)], user_loc)

    avs = [[act_loc[a, pl.ds(g * 16, 16)] for g in range(NG)]
           for a in range(ACT_V)]
    for half in range(2):
        def build_p(pp, carry):
            p = half * (PPT // 2) + pp
            for g in range(NG):
                sl = pl.ds(g * 16, 16)
                pv = pos_loc[p, sl]
                for a in range(ACT_V):
                    rows[0, pp * ACT_V + a, sl] = pv + avs[a][g]
            return carry

        lax.fori_loop(0, PPT // 2, build_p, 0)
        pltpu.sync_copy(rows.at[0, pl.ds(0, HPT)],
                        combo_sh.at[pl.ds(sid * CPT + half * HPT, HPT)])
    plsc.subcore_barrier()

    isems = (isem0, isem1)
    csems = (csem0, csem1)
    osems = (osem0, osem1)

    def stage_ids(c):
        e0 = (base_b + c * CB) * L
        cpar = lax.rem(c, 2)
        pltpu.sync_copy(item_ids.at[pl.ds(e0, CB * L)], iid_loc.at[cpar])
        pltpu.sync_copy(pos_ids.at[pl.ds(e0, CB * L)], pid_loc.at[cpar])
        pltpu.sync_copy(act_ids.at[pl.ds(e0, CB * L)], aid_loc.at[cpar])

    def prep_cidx(b, par):
        cpar = lax.rem(b // CB, 2)
        r0 = lax.rem(b, CB) * L
        for g16 in range(L // 16):
            sl_i = pl.ds(r0 + g16 * 16, 16)
            sl_o = pl.ds(g16 * 16, 16)
            cidx[par, sl_o] = pid_loc[cpar, sl_i] * ACT_V + aid_loc[cpar, sl_i]
        cidx[par, pl.ds(L - 16, 16)] = (pid_loc[cpar, pl.ds(r0 + L - 16, 16)]
                                        * ACT_V
                                        + aid_loc[cpar, pl.ds(r0 + L - 16, 16)])

    def gathers(b, par):
        cpar = lax.rem(b // CB, 2)
        r0 = lax.rem(b, CB) * L
        pltpu.async_copy(item_tab.at[iid_loc.at[cpar, pl.ds(r0, G0)]],
                         rows.at[par, pl.ds(0, G0)], isems[par])
        pltpu.async_copy(item_tab.at[iid_loc.at[cpar, pl.ds(r0 + G0, G1)]],
                         rows.at[par, pl.ds(G0, G1)], isems[par])
        pltpu.async_copy(combo_sh.at[cidx.at[par, pl.ds(0, G0)]],
                         combo.at[par, pl.ds(0, G0)], csems[par])
        pltpu.async_copy(combo_sh.at[cidx.at[par, pl.ds(G0, G1)]],
                         combo.at[par, pl.ds(G0, G1)], csems[par])

    def wait_gathers(b, par):
        cpar = lax.rem(b // CB, 2)
        r0 = lax.rem(b, CB) * L
        pltpu.make_async_copy(item_tab.at[iid_loc.at[cpar, pl.ds(r0, G0)]],
                              rows.at[par, pl.ds(0, G0)], isems[par]).wait()
        pltpu.make_async_copy(item_tab.at[iid_loc.at[cpar, pl.ds(r0 + G0, G1)]],
                              rows.at[par, pl.ds(G0, G1)], isems[par]).wait()
        pltpu.make_async_copy(combo_sh.at[cidx.at[par, pl.ds(0, G0)]],
                              combo.at[par, pl.ds(0, G0)], csems[par]).wait()
        pltpu.make_async_copy(combo_sh.at[cidx.at[par, pl.ds(G0, G1)]],
                              combo.at[par, pl.ds(G0, G1)], csems[par]).wait()

    def issue_out(b, par):
        pltpu.async_copy(ostage.at[par], out_hbm.at[pl.ds((base_b + b) * ORPB,
                                                          ORPB)], osems[par])

    def wait_out(b, par):
        pltpu.make_async_copy(ostage.at[par],
                              out_hbm.at[pl.ds((base_b + b) * ORPB, ORPB)],
                              osems[par]).wait()

    def compute(b, par):
        lb = b
        uvs = [user_loc[lb, pl.ds(g * 16, 16)] for g in range(NG)]

        def addrow(i, carry):
            for j in range(4):
                r = i * 4 + j
                orow = 2 * i + (j // 2)
                oc0 = (j % 2) * D
                for g in range(NG):
                    sl = pl.ds(g * 16, 16)
                    ostage[par, orow, pl.ds(oc0 + g * 16, 16)] = (
                        rows[par, r, sl] + combo[par, r, sl] + uvs[g])
            return carry

        lax.fori_loop(0, L // 4, addrow, 0)

    stage_ids(0)
    prep_cidx(0, 0)
    gathers(0, 0)

    def loop_body(k, carry):
        for par in range(2):
            b = 2 * k + par
            wait_gathers(b, par)
            nxt = b + 1

            @pl.when(nxt < BPW)
            def _prep():
                @pl.when(lax.rem(nxt, CB) == 0)
                def _stage():
                    stage_ids(nxt // CB)
                prep_cidx(nxt, 1 - par)
                gathers(nxt, 1 - par)

            @pl.when(b >= 2)
            def _wout():
                wait_out(b - 2, par)

            compute(b, par)
            issue_out(b, par)
        return carry

    lax.fori_loop(0, BPW // 2, loop_body, 0)
    wait_out(BPW - 2, 0)
    wait_out(BPW - 1, 1)


@jax.jit
def _run(item_ids, pos_ids, act_ids, user_ids,
         item_tab, user_tab, pos_tab, act_tab):
    mesh = plsc.VectorSubcoreMesh(core_axis_name="c", subcore_axis_name="s")
    f = functools.partial(
        pl.kernel,
        out_type=jax.ShapeDtypeStruct((B * L * D // OW, OW), jnp.float32),
        mesh=mesh,
        compiler_params=pltpu.CompilerParams(use_tc_tiling_on_sc=False),
        scratch_types=[
            pltpu.VMEM_SHARED((CV_PAD, D), jnp.float32),
            pltpu.VMEM((PPT, D), jnp.float32),
            pltpu.VMEM((ACT_V, D), jnp.float32),
            pltpu.VMEM((BPW, D), jnp.float32),
            pltpu.VMEM((2, CB * L), jnp.int32),
            pltpu.VMEM((2, CB * L), jnp.int32),
            pltpu.VMEM((2, CB * L), jnp.int32),
            pltpu.VMEM((2, L), jnp.int32),
            pltpu.VMEM((2, L, D), jnp.float32),
            pltpu.VMEM((2, L, D), jnp.float32),
            pltpu.VMEM((2, ORPB, OW), jnp.float32),
            pltpu.SemaphoreType.DMA,
            pltpu.SemaphoreType.DMA,
            pltpu.SemaphoreType.DMA,
            pltpu.SemaphoreType.DMA,
            pltpu.SemaphoreType.DMA,
            pltpu.SemaphoreType.DMA,
            pltpu.SemaphoreType.DMA,
        ],
    )(_sc_body)
    pos_pad = jnp.zeros((NS * PPT, D), jnp.float32).at[:POS_V].set(pos_tab)
    user_rows = jnp.take(user_tab, user_ids, axis=0)
    out = f(item_ids, pos_ids, act_ids, user_rows,
            item_tab, pos_pad, act_tab)
    return out.reshape(B, L, D)


def kernel(item_ids, pos_ids, action_ids, user_ids,
           item_table, user_table, pos_table, action_table):
    return _run(item_ids.astype(jnp.int32).reshape(-1),
                pos_ids.astype(jnp.int32).reshape(-1),
                action_ids.astype(jnp.int32).reshape(-1),
                user_ids.astype(jnp.int32),
                item_table, user_table, pos_table, action_table)


# speedup vs baseline: 1.7746x; 1.0014x over previous
import functools

import jax
import jax.numpy as jnp
from jax import lax
from jax.experimental import pallas as pl
from jax.experimental.pallas import tpu as pltpu
from jax.experimental.pallas import tpu_sc as plsc

B, L, D = 4096, 200, 64
POS_V = 2 * 200 + 1
ACT_V = 8 + 1
NC, NS = 2, 16
NW = NC * NS
BPW = B // NW
CB = 8
G0, G1 = 104, 96
NG = D // 16
PPT = 26
CPT = PPT * ACT_V
CV_PAD = NS * CPT
HPT = CPT // 2
OW = 128
ORPB = L * D // OW


def _sc_body(item_ids, pos_ids, act_ids, user_rows,
             item_tab, pos_tab, act_tab,
             out_hbm,
             combo_sh, pos_loc, act_loc, user_loc,
             iid_loc, pid_loc, aid_loc, cidx, rows, combo, ostage,
             isem0, isem1, csem0, csem1, osem0, osem1, ssem):
    cid = lax.axis_index("c")
    sid = lax.axis_index("s")
    wid = sid * NC + cid
    base_b = wid * BPW

    pltpu.sync_copy(pos_tab.at[pl.ds(sid * PPT, PPT)], pos_loc)
    pltpu.sync_copy(act_tab, act_loc)
    pltpu.sync_copy(user_rows.at[pl.ds(base_b, BPW)], user_loc)

    avs = [[act_loc[a, pl.ds(g * 16, 16)] for g in range(NG)]
           for a in range(ACT_V)]
    for half in range(2):
        def build_p(pp, carry):
            p = half * (PPT // 2) + pp
            for g in range(NG):
                sl = pl.ds(g * 16, 16)
                pv = pos_loc[p, sl]
                for a in range(ACT_V):
                    rows[0, pp * ACT_V + a, sl] = pv + avs[a][g]
            return carry

        lax.fori_loop(0, PPT // 2, build_p, 0)
        pltpu.sync_copy(rows.at[0, pl.ds(0, HPT)],
                        combo_sh.at[pl.ds(sid * CPT + half * HPT, HPT)])
    plsc.subcore_barrier()

    isems = (isem0, isem1)
    csems = (csem0, csem1)
    osems = (osem0, osem1)

    def stage_ids(c):
        b0c = base_b + c * CB
        cpar = lax.rem(c, 2)
        pltpu.sync_copy(item_ids.at[pl.ds(b0c, CB)], iid_loc.at[cpar])
        pltpu.sync_copy(pos_ids.at[pl.ds(b0c, CB)], pid_loc.at[cpar])
        pltpu.sync_copy(act_ids.at[pl.ds(b0c, CB)], aid_loc.at[cpar])

    def prep_cidx(b, par):
        cpar = lax.rem(b // CB, 2)
        bb = lax.rem(b, CB)
        for g16 in range(L // 16):
            sl = pl.ds(g16 * 16, 16)
            cidx[par, sl] = (pid_loc[cpar, bb, sl] * ACT_V
                             + aid_loc[cpar, bb, sl])
        sl = pl.ds(L - 16, 16)
        cidx[par, sl] = pid_loc[cpar, bb, sl] * ACT_V + aid_loc[cpar, bb, sl]

    def gathers(b, par):
        cpar = lax.rem(b // CB, 2)
        bb = lax.rem(b, CB)
        pltpu.async_copy(item_tab.at[iid_loc.at[cpar, bb, pl.ds(0, G0)]],
                         rows.at[par, pl.ds(0, G0)], isems[par])
        pltpu.async_copy(item_tab.at[iid_loc.at[cpar, bb, pl.ds(G0, G1)]],
                         rows.at[par, pl.ds(G0, G1)], isems[par])
        pltpu.async_copy(combo_sh.at[cidx.at[par, pl.ds(0, G0)]],
                         combo.at[par, pl.ds(0, G0)], csems[par])
        pltpu.async_copy(combo_sh.at[cidx.at[par, pl.ds(G0, G1)]],
                         combo.at[par, pl.ds(G0, G1)], csems[par])

    def wait_gathers(b, par):
        cpar = lax.rem(b // CB, 2)
        bb = lax.rem(b, CB)
        pltpu.make_async_copy(item_tab.at[iid_loc.at[cpar, bb, pl.ds(0, G0)]],
                              rows.at[par, pl.ds(0, G0)], isems[par]).wait()
        pltpu.make_async_copy(item_tab.at[iid_loc.at[cpar, bb, pl.ds(G0, G1)]],
                              rows.at[par, pl.ds(G0, G1)], isems[par]).wait()
        pltpu.make_async_copy(combo_sh.at[cidx.at[par, pl.ds(0, G0)]],
                              combo.at[par, pl.ds(0, G0)], csems[par]).wait()
        pltpu.make_async_copy(combo_sh.at[cidx.at[par, pl.ds(G0, G1)]],
                              combo.at[par, pl.ds(G0, G1)], csems[par]).wait()

    def issue_out(b, par):
        pltpu.async_copy(ostage.at[par], out_hbm.at[pl.ds((base_b + b) * ORPB,
                                                          ORPB)], osems[par])

    def wait_out(b, par):
        pltpu.make_async_copy(ostage.at[par],
                              out_hbm.at[pl.ds((base_b + b) * ORPB, ORPB)],
                              osems[par]).wait()

    def compute(b, par):
        lb = b
        uvs = [user_loc[lb, pl.ds(g * 16, 16)] for g in range(NG)]

        def addrow(i, carry):
            for j in range(4):
                r = i * 4 + j
                orow = 2 * i + (j // 2)
                oc0 = (j % 2) * D
                for g in range(NG):
                    sl = pl.ds(g * 16, 16)
                    ostage[par, orow, pl.ds(oc0 + g * 16, 16)] = (
                        rows[par, r, sl] + combo[par, r, sl] + uvs[g])
            return carry

        lax.fori_loop(0, L // 4, addrow, 0)

    stage_ids(0)
    prep_cidx(0, 0)
    gathers(0, 0)

    def loop_body(k, carry):
        for par in range(2):
            b = 2 * k + par
            wait_gathers(b, par)
            nxt = b + 1

            @pl.when(nxt < BPW)
            def _prep():
                @pl.when(lax.rem(nxt, CB) == 0)
                def _stage():
                    stage_ids(nxt // CB)
                prep_cidx(nxt, 1 - par)
                gathers(nxt, 1 - par)

            @pl.when(b >= 2)
            def _wout():
                wait_out(b - 2, par)

            compute(b, par)
            issue_out(b, par)
        return carry

    lax.fori_loop(0, BPW // 2, loop_body, 0)
    wait_out(BPW - 2, 0)
    wait_out(BPW - 1, 1)


@jax.jit
def _run(item_ids, pos_ids, act_ids, user_ids,
         item_tab, user_tab, pos_tab, act_tab):
    mesh = plsc.VectorSubcoreMesh(core_axis_name="c", subcore_axis_name="s")
    f = functools.partial(
        pl.kernel,
        out_type=jax.ShapeDtypeStruct((B * L * D // OW, OW), jnp.float32),
        mesh=mesh,
        compiler_params=pltpu.CompilerParams(use_tc_tiling_on_sc=False),
        scratch_types=[
            pltpu.VMEM_SHARED((CV_PAD, D), jnp.float32),
            pltpu.VMEM((PPT, D), jnp.float32),
            pltpu.VMEM((ACT_V, D), jnp.float32),
            pltpu.VMEM((BPW, D), jnp.float32),
            pltpu.VMEM((2, CB, L), jnp.int32),
            pltpu.VMEM((2, CB, L), jnp.int32),
            pltpu.VMEM((2, CB, L), jnp.int32),
            pltpu.VMEM((2, L), jnp.int32),
            pltpu.VMEM((2, L, D), jnp.float32),
            pltpu.VMEM((2, L, D), jnp.float32),
            pltpu.VMEM((2, ORPB, OW), jnp.float32),
            pltpu.SemaphoreType.DMA,
            pltpu.SemaphoreType.DMA,
            pltpu.SemaphoreType.DMA,
            pltpu.SemaphoreType.DMA,
            pltpu.SemaphoreType.DMA,
            pltpu.SemaphoreType.DMA,
            pltpu.SemaphoreType.DMA,
        ],
    )(_sc_body)
    pos_pad = jnp.zeros((NS * PPT, D), jnp.float32).at[:POS_V].set(pos_tab)
    user_rows = jnp.take(user_tab, user_ids, axis=0)
    out = f(item_ids, pos_ids, act_ids, user_rows,
            item_tab, pos_pad, act_tab)
    return out.reshape(B, L, D)


def kernel(item_ids, pos_ids, action_ids, user_ids,
           item_table, user_table, pos_table, action_table):
    return _run(item_ids.astype(jnp.int32),
                pos_ids.astype(jnp.int32),
                action_ids.astype(jnp.int32),
                user_ids.astype(jnp.int32),
                item_table, user_table, pos_table, action_table)


# speedup vs baseline: 1.8348x; 1.0339x over previous
import functools

import jax
import jax.numpy as jnp
from jax import lax
from jax.experimental import pallas as pl
from jax.experimental.pallas import tpu as pltpu
from jax.experimental.pallas import tpu_sc as plsc

B, L, D = 4096, 200, 64
POS_V = 2 * 200 + 1
ACT_V = 8 + 1
NC, NS = 2, 16
NW = NC * NS
BPW = B // NW
CB = 16
G0, G1 = 104, 96
NG = D // 16
PPT = 26
CPT = PPT * ACT_V
CV_PAD = NS * CPT
HPT = CPT // 2
OW = 128
ORPB = L * D // OW


def _sc_body(item_ids, pos_ids, act_ids, user_rows,
             item_tab, pos_tab, act_tab,
             out_hbm,
             combo_sh, pos_loc, act_loc, user_loc,
             iid_loc, pid_loc, aid_loc, cidx, rows, combo, ostage,
             isem0, isem1, csem0, csem1, osem0, osem1, ssem):
    cid = lax.axis_index("c")
    sid = lax.axis_index("s")
    wid = sid * NC + cid
    base_b = wid * BPW

    pltpu.sync_copy(pos_tab.at[pl.ds(sid * PPT, PPT)], pos_loc)
    pltpu.sync_copy(act_tab, act_loc)
    pltpu.sync_copy(user_rows.at[pl.ds(base_b, BPW)], user_loc)

    avs = [[act_loc[a, pl.ds(g * 16, 16)] for g in range(NG)]
           for a in range(ACT_V)]
    for half in range(2):
        def build_p(pp, carry):
            p = half * (PPT // 2) + pp
            for g in range(NG):
                sl = pl.ds(g * 16, 16)
                pv = pos_loc[p, sl]
                for a in range(ACT_V):
                    rows[0, pp * ACT_V + a, sl] = pv + avs[a][g]
            return carry

        lax.fori_loop(0, PPT // 2, build_p, 0)
        pltpu.sync_copy(rows.at[0, pl.ds(0, HPT)],
                        combo_sh.at[pl.ds(sid * CPT + half * HPT, HPT)])
    plsc.subcore_barrier()

    isems = (isem0, isem1)
    csems = (csem0, csem1)
    osems = (osem0, osem1)

    def _stage_copies(c):
        b0c = base_b + c * CB
        cpar = lax.rem(c, 2)
        return (
            (item_ids.at[pl.ds(b0c, CB)], iid_loc.at[cpar]),
            (pos_ids.at[pl.ds(b0c, CB)], pid_loc.at[cpar]),
            (act_ids.at[pl.ds(b0c, CB)], aid_loc.at[cpar]),
        )

    def stage_ids_sync(c):
        for src, dst in _stage_copies(c):
            pltpu.sync_copy(src, dst)

    def stage_ids_issue(c):
        for src, dst in _stage_copies(c):
            pltpu.async_copy(src, dst, ssem)

    def stage_ids_wait(c):
        for src, dst in _stage_copies(c):
            pltpu.make_async_copy(src, dst, ssem).wait()

    def prep_cidx(b, par):
        cpar = lax.rem(b // CB, 2)
        bb = lax.rem(b, CB)
        for g16 in range(L // 16):
            sl = pl.ds(g16 * 16, 16)
            cidx[par, sl] = (pid_loc[cpar, bb, sl] * ACT_V
                             + aid_loc[cpar, bb, sl])
        sl = pl.ds(L - 16, 16)
        cidx[par, sl] = pid_loc[cpar, bb, sl] * ACT_V + aid_loc[cpar, bb, sl]

    def gathers(b, par):
        cpar = lax.rem(b // CB, 2)
        bb = lax.rem(b, CB)
        pltpu.async_copy(item_tab.at[iid_loc.at[cpar, bb, pl.ds(0, G0)]],
                         rows.at[par, pl.ds(0, G0)], isems[par])
        pltpu.async_copy(item_tab.at[iid_loc.at[cpar, bb, pl.ds(G0, G1)]],
                         rows.at[par, pl.ds(G0, G1)], isems[par])
        pltpu.async_copy(combo_sh.at[cidx.at[par, pl.ds(0, G0)]],
                         combo.at[par, pl.ds(0, G0)], csems[par])
        pltpu.async_copy(combo_sh.at[cidx.at[par, pl.ds(G0, G1)]],
                         combo.at[par, pl.ds(G0, G1)], csems[par])

    def wait_gathers(b, par):
        cpar = lax.rem(b // CB, 2)
        bb = lax.rem(b, CB)
        pltpu.make_async_copy(item_tab.at[iid_loc.at[cpar, bb, pl.ds(0, G0)]],
                              rows.at[par, pl.ds(0, G0)], isems[par]).wait()
        pltpu.make_async_copy(item_tab.at[iid_loc.at[cpar, bb, pl.ds(G0, G1)]],
                              rows.at[par, pl.ds(G0, G1)], isems[par]).wait()
        pltpu.make_async_copy(combo_sh.at[cidx.at[par, pl.ds(0, G0)]],
                              combo.at[par, pl.ds(0, G0)], csems[par]).wait()
        pltpu.make_async_copy(combo_sh.at[cidx.at[par, pl.ds(G0, G1)]],
                              combo.at[par, pl.ds(G0, G1)], csems[par]).wait()

    def issue_out(b, par):
        pltpu.async_copy(ostage.at[par], out_hbm.at[pl.ds((base_b + b) * ORPB,
                                                          ORPB)], osems[par])

    def wait_out(b, par):
        pltpu.make_async_copy(ostage.at[par],
                              out_hbm.at[pl.ds((base_b + b) * ORPB, ORPB)],
                              osems[par]).wait()

    def compute(b, par):
        lb = b
        uvs = [user_loc[lb, pl.ds(g * 16, 16)] for g in range(NG)]

        def addrow(i, carry):
            for j in range(4):
                r = i * 4 + j
                orow = 2 * i + (j // 2)
                oc0 = (j % 2) * D
                for g in range(NG):
                    sl = pl.ds(g * 16, 16)
                    ostage[par, orow, pl.ds(oc0 + g * 16, 16)] = (
                        rows[par, r, sl] + combo[par, r, sl] + uvs[g])
            return carry

        lax.fori_loop(0, L // 4, addrow, 0)

    stage_ids_sync(0)
    stage_ids_issue(1)
    prep_cidx(0, 0)
    gathers(0, 0)

    def loop_body(k, carry):
        for par in range(2):
            b = 2 * k + par
            wait_gathers(b, par)
            nxt = b + 1

            @pl.when(nxt < BPW)
            def _prep():
                @pl.when(lax.rem(nxt, CB) == 0)
                def _stage():
                    stage_ids_wait(nxt // CB)
                prep_cidx(nxt, 1 - par)
                gathers(nxt, 1 - par)

                @pl.when((lax.rem(nxt, CB) == CB // 2)
                         & (nxt // CB + 1 < BPW // CB))
                def _prefetch():
                    stage_ids_issue(nxt // CB + 1)

            @pl.when(b >= 2)
            def _wout():
                wait_out(b - 2, par)

            compute(b, par)
            issue_out(b, par)
        return carry

    lax.fori_loop(0, BPW // 2, loop_body, 0)
    wait_out(BPW - 2, 0)
    wait_out(BPW - 1, 1)


@jax.jit
def _run(item_ids, pos_ids, act_ids, user_ids,
         item_tab, user_tab, pos_tab, act_tab):
    mesh = plsc.VectorSubcoreMesh(core_axis_name="c", subcore_axis_name="s")
    f = functools.partial(
        pl.kernel,
        out_type=jax.ShapeDtypeStruct((B * L * D // OW, OW), jnp.float32),
        mesh=mesh,
        compiler_params=pltpu.CompilerParams(use_tc_tiling_on_sc=False),
        scratch_types=[
            pltpu.VMEM_SHARED((CV_PAD, D), jnp.float32),
            pltpu.VMEM((PPT, D), jnp.float32),
            pltpu.VMEM((ACT_V, D), jnp.float32),
            pltpu.VMEM((BPW, D), jnp.float32),
            pltpu.VMEM((2, CB, L), jnp.int32),
            pltpu.VMEM((2, CB, L), jnp.int32),
            pltpu.VMEM((2, CB, L), jnp.int32),
            pltpu.VMEM((2, L), jnp.int32),
            pltpu.VMEM((2, L, D), jnp.float32),
            pltpu.VMEM((2, L, D), jnp.float32),
            pltpu.VMEM((2, ORPB, OW), jnp.float32),
            pltpu.SemaphoreType.DMA,
            pltpu.SemaphoreType.DMA,
            pltpu.SemaphoreType.DMA,
            pltpu.SemaphoreType.DMA,
            pltpu.SemaphoreType.DMA,
            pltpu.SemaphoreType.DMA,
            pltpu.SemaphoreType.DMA,
        ],
    )(_sc_body)
    pos_pad = jnp.zeros((NS * PPT, D), jnp.float32).at[:POS_V].set(pos_tab)
    user_rows = jnp.take(user_tab, user_ids, axis=0)
    out = f(item_ids, pos_ids, act_ids, user_rows,
            item_tab, pos_pad, act_tab)
    return out.reshape(B, L, D)


def kernel(item_ids, pos_ids, action_ids, user_ids,
           item_table, user_table, pos_table, action_table):
    return _run(item_ids.astype(jnp.int32),
                pos_ids.astype(jnp.int32),
                action_ids.astype(jnp.int32),
                user_ids.astype(jnp.int32),
                item_table, user_table, pos_table, action_table)
